# Initial kernel scaffold; baseline (speedup 1.0000x reference)
import jax
import jax.numpy as jnp
from jax.experimental import pallas as pl


def kernel(x, edge_attr, edge_index, coords, bc_disp, bc_rot, params):
    raise NotImplementedError("write your pallas kernel here")


# speedup vs baseline: 3.0885x; 3.0885x over previous
import functools

import jax
import jax.numpy as jnp
from jax import lax
from jax.experimental import pallas as pl
from jax.experimental.pallas import tpu as pltpu
from jax.experimental.pallas import tpu_sc as plsc

H = 64
NL = 6
NN = 10000
NE = 320000


def _dot(a, b):
    return jnp.dot(a, b, precision=jax.lax.Precision.HIGHEST)

_NODE_BLK = 1000
_EDGE_BLK = 3200


def _node_enc_body(x_ref, w1_ref, b1_ref, w2_ref, b2_ref, a_ref, b_ref,
                   h_ref, p_ref, q_ref):
    t = jnp.maximum(_dot(x_ref[...], w1_ref[...]) + b1_ref[...], 0.0)
    h = _dot(t, w2_ref[...]) + b2_ref[...]
    h_ref[...] = h
    p_ref[...] = _dot(h, a_ref[...])
    q_ref[...] = _dot(h, b_ref[...])


def _node_encode(x16, w1, b1, w2, b2, a0, b0):
    grid = NN // _NODE_BLK
    blk = lambda r, c: pl.BlockSpec((r, c), lambda i: (i, 0))
    wspec = lambda r, c: pl.BlockSpec((r, c), lambda i: (0, 0))
    out = jax.ShapeDtypeStruct((NN, H), jnp.float32)
    return pl.pallas_call(
        _node_enc_body,
        grid=(grid,),
        in_specs=[blk(_NODE_BLK, 16), wspec(16, H), wspec(1, H), wspec(H, H),
                  wspec(1, H), wspec(H, H), wspec(H, H)],
        out_specs=[blk(_NODE_BLK, H)] * 3,
        out_shape=[out, out, out],
    )(x16, w1, b1, w2, b2, a0, b0)


def _edge_enc_body(ea_ref, w1_ref, b1_ref, w2_ref, b2_ref, c_ref, be_ref,
                   *e_refs):
    t = jnp.maximum(_dot(ea_ref[...], w1_ref[...]) + b1_ref[...], 0.0)
    e = _dot(t, w2_ref[...]) + b2_ref[...]
    for l in range(NL):
        e_refs[l][...] = _dot(e, c_ref[l]) + be_ref[l:l + 1, :]


def _edge_encode(ea16, w1, b1, w2, b2, c_all, be_all):
    grid = NE // _EDGE_BLK
    blk = lambda r, c: pl.BlockSpec((r, c), lambda i: (i, 0))
    wspec = lambda r, c: pl.BlockSpec((r, c), lambda i: (0, 0))
    out = jax.ShapeDtypeStruct((NE, H), jnp.float32)
    return pl.pallas_call(
        _edge_enc_body,
        grid=(grid,),
        in_specs=[blk(_EDGE_BLK, 16), wspec(16, H), wspec(1, H), wspec(H, H),
                  wspec(1, H),
                  pl.BlockSpec((NL, H, H), lambda i: (0, 0, 0)),
                  wspec(NL, H)],
        out_specs=[blk(_EDGE_BLK, H)] * NL,
        out_shape=[out] * NL,
    )(ea16, w1, b1, w2, b2, c_all, be_all)


def _node_upd_body(h_ref, g0_ref, g1_ref, wh_ref, wa_ref, bn_ref, a_ref,
                   b_ref, hn_ref, p_ref, q_ref):
    h = h_ref[...]
    agg = g0_ref[...] + g1_ref[...]
    u = jnp.maximum(_dot(h, wh_ref[...]) + _dot(agg, wa_ref[...]) + bn_ref[...], 0.0)
    hn = h + u
    hn_ref[...] = hn
    p_ref[...] = _dot(hn, a_ref[...])
    q_ref[...] = _dot(hn, b_ref[...])


def _node_update(h, g0, g1, wh, wa, bn, a_next, b_next):
    grid = NN // _NODE_BLK
    blk = lambda r, c: pl.BlockSpec((r, c), lambda i: (i, 0))
    wspec = lambda r, c: pl.BlockSpec((r, c), lambda i: (0, 0))
    out = jax.ShapeDtypeStruct((NN, H), jnp.float32)
    return pl.pallas_call(
        _node_upd_body,
        grid=(grid,),
        in_specs=[blk(_NODE_BLK, H)] * 3 + [wspec(H, H), wspec(H, H),
                                            wspec(1, H), wspec(H, H),
                                            wspec(H, H)],
        out_specs=[blk(_NODE_BLK, H)] * 3,
        out_shape=[out, out, out],
    )(h, g0, g1, wh, wa, bn, a_next, b_next)


def _minmax_body(c_ref, o_ref):
    c = c_ref[...]
    o_ref[0:1, :] = jnp.min(c, axis=0, keepdims=True)
    o_ref[1:2, :] = jnp.max(c, axis=0, keepdims=True)


def _coords_minmax(coords8):
    return pl.pallas_call(
        _minmax_body,
        out_shape=jax.ShapeDtypeStruct((2, 8), jnp.float32),
    )(coords8)


def _dec_body(c_ref, mm_ref, h_ref, bcd_ref, bcr_ref, w1c_ref, w1h_ref,
              b1_ref, w2_ref, b2_ref, w3_ref, b3_ref, w4_ref, b4_ref, o_ref):
    cmin = mm_ref[0:1, :]
    crange = jnp.maximum(mm_ref[1:2, :] - cmin, 1e-8)
    cn = (c_ref[...] - cmin) / crange
    t = jnp.maximum(_dot(cn, w1c_ref[...]) + _dot(h_ref[...], w1h_ref[...])
                    + b1_ref[...], 0.0)
    t = jnp.maximum(_dot(t, w2_ref[...]) + b2_ref[...], 0.0)
    t = jnp.maximum(_dot(t, w3_ref[...]) + b3_ref[...], 0.0)
    pred = _dot(t, w4_ref[...]) + b4_ref[...]
    col = lax.broadcasted_iota(jnp.int32, pred.shape, 1)
    factor = jnp.where(col < 2, 1.0 - bcd_ref[...], 1.0 - bcr_ref[...])
    o_ref[...] = pred * factor


def _decode(coords8, mm, h, bcd, bcr, w1c, w1h, b1, w2, b2, w3, b3, w4, b4):
    grid = NN // _NODE_BLK
    blk = lambda r, c: pl.BlockSpec((r, c), lambda i: (i, 0))
    wspec = lambda r, c: pl.BlockSpec((r, c), lambda i: (0, 0))
    return pl.pallas_call(
        _dec_body,
        grid=(grid,),
        in_specs=[blk(_NODE_BLK, 8), wspec(2, 8), blk(_NODE_BLK, H),
                  blk(_NODE_BLK, 1), blk(_NODE_BLK, 1), wspec(8, H),
                  wspec(H, H), wspec(1, H), wspec(H, H), wspec(1, H),
                  wspec(H, H), wspec(1, H), wspec(H, 8), wspec(1, 8)],
        out_specs=blk(_NODE_BLK, 8),
        out_shape=jax.ShapeDtypeStruct((NN, 8), jnp.float32),
    )(coords8, mm, h, bcd, bcr, w1c, w1h, b1, w2, b2, w3, b3, w4, b4)


_NC = 2
_NS = 16
_NT = _NC * _NS
_K = 128
_NCHUNK = NE // _K
_CPT = -(-_NCHUNK // _NT)
_NNP = 10240
_RPT = _NNP // _NS
_ZR = 128


def _mp_body(p_hbm, q_hbm, e_hbm, src_hbm, dst_hbm, out_hbm,
             idx_s, idx_d, pbuf, qbuf, ebuf, zbuf, agg_sh, sem_p, sem_q):
    cid = lax.axis_index("c")
    sid = lax.axis_index("s")
    wid = sid * _NC + cid

    zvec = jnp.zeros((16,), jnp.float32)

    def zrow(r, carry):
        for g in range(H // 16):
            zbuf[r, pl.ds(g * 16, 16)] = zvec
        return carry

    lax.fori_loop(0, _ZR, zrow, 0)
    for j in range(_RPT // _ZR):
        pltpu.sync_copy(zbuf, agg_sh.at[pl.ds(sid * _RPT + j * _ZR, _ZR)])
    plsc.subcore_barrier()

    def chunk(j, carry):
        c = wid + _NT * j

        @pl.when(c < _NCHUNK)
        def _():
            base = c * _K
            pltpu.sync_copy(src_hbm.at[pl.ds(base, _K)], idx_s)
            pltpu.sync_copy(dst_hbm.at[pl.ds(base, _K)], idx_d)
            cp_p = pltpu.make_async_copy(p_hbm.at[idx_s], pbuf, sem_p)
            cp_p.start()
            cp_q = pltpu.make_async_copy(q_hbm.at[idx_d], qbuf, sem_q)
            cp_q.start()
            pltpu.sync_copy(e_hbm.at[pl.ds(base, _K)], ebuf)
            cp_p.wait()
            cp_q.wait()

            def row(r, rc):
                for g in range(H // 16):
                    s = pl.ds(g * 16, 16)
                    ebuf[r, s] = jnp.maximum(
                        pbuf[r, s] + qbuf[r, s] + ebuf[r, s], 0.0)
                return rc

            lax.fori_loop(0, _K, row, 0)
            pltpu.sync_copy(ebuf, agg_sh.at[idx_d], add=True)

        return carry

    lax.fori_loop(0, _CPT, chunk, 0)
    plsc.subcore_barrier()
    pltpu.sync_copy(agg_sh.at[pl.ds(sid * _RPT, _RPT)],
                    out_hbm.at[cid, pl.ds(sid * _RPT, _RPT)])


@functools.cache
def _build_mp_call():
    return pl.kernel(
        _mp_body,
        out_type=jax.ShapeDtypeStruct((_NC, _NNP, H), jnp.float32),
        mesh=plsc.VectorSubcoreMesh(core_axis_name="c", subcore_axis_name="s"),
        compiler_params=pltpu.CompilerParams(use_tc_tiling_on_sc=False),
        scratch_types=[
            pltpu.VMEM((_K,), jnp.int32),
            pltpu.VMEM((_K,), jnp.int32),
            pltpu.VMEM((_K, H), jnp.float32),
            pltpu.VMEM((_K, H), jnp.float32),
            pltpu.VMEM((_K, H), jnp.float32),
            pltpu.VMEM((_ZR, H), jnp.float32),
            pltpu.VMEM_SHARED((_NNP, H), jnp.float32),
            pltpu.SemaphoreType.DMA,
            pltpu.SemaphoreType.DMA,
        ],
    )


def kernel(x, edge_attr, edge_index, coords, bc_disp, bc_rot, params):
    p = params
    src = edge_index[0]
    dst = edge_index[1]

    x16 = jnp.pad(x, ((0, 0), (0, 16 - x.shape[1])))
    ne_w1 = jnp.pad(p['ne_W1'], ((0, 16 - p['ne_W1'].shape[0]), (0, 0)))
    ea16 = jnp.pad(edge_attr, ((0, 0), (0, 16 - edge_attr.shape[1])))
    ee_w1 = jnp.pad(p['ee_W1'], ((0, 16 - p['ee_W1'].shape[0]), (0, 0)))

    we = p['mp_We']
    a_all = we[:, 0:H, :]
    b_all = we[:, H:2 * H, :]
    c_all = we[:, 2 * H:3 * H, :]
    wn = p['mp_Wn']
    wh_all = wn[:, 0:H, :]
    wa_all = wn[:, H:2 * H, :]
    be_all = p['mp_be']
    bn_all = p['mp_bn']

    r1 = lambda v: v.reshape(1, -1)

    h, pproj, qproj = _node_encode(x16, ne_w1, r1(p['ne_b1']), p['ne_W2'],
                                   r1(p['ne_b2']), a_all[0], b_all[0])
    e_layers = _edge_encode(ea16, ee_w1, r1(p['ee_b1']), p['ee_W2'],
                            r1(p['ee_b2']), c_all, be_all)

    mp_call = _build_mp_call()
    for l in range(NL):
        agg2 = mp_call(pproj, qproj, e_layers[l], src, dst)
        nxt = (l + 1) % NL
        h, pproj, qproj = _node_update(h, agg2[0, :NN], agg2[1, :NN], wh_all[l],
                                       wa_all[l], r1(bn_all[l]), a_all[nxt],
                                       b_all[nxt])

    coords8 = jnp.pad(coords, ((0, 0), (0, 8 - coords.shape[1])))
    mm = _coords_minmax(coords8)
    w1c = jnp.pad(p['dec_W1'][0:3, :], ((0, 5), (0, 0)))
    w1h = p['dec_W1'][3:, :]
    w4 = jnp.pad(p['dec_W4'], ((0, 0), (0, 8 - p['dec_W4'].shape[1])))
    b4 = jnp.pad(r1(p['dec_b4']), ((0, 0), (0, 8 - p['dec_b4'].shape[0])))
    pred8 = _decode(coords8, mm, h, bc_disp, bc_rot, w1c, w1h,
                    r1(p['dec_b1']), p['dec_W2'], r1(p['dec_b2']),
                    p['dec_W3'], r1(p['dec_b3']), w4, b4)
    return pred8[:, 0:3]


# speedup vs baseline: 3.5307x; 1.1432x over previous
import functools

import jax
import jax.numpy as jnp
from jax import lax
from jax.experimental import pallas as pl
from jax.experimental.pallas import tpu as pltpu
from jax.experimental.pallas import tpu_sc as plsc

H = 64
NL = 6
NN = 10000
NE = 320000


def _dot(a, b):
    return jnp.dot(a, b, precision=jax.lax.Precision.HIGHEST)

_NODE_BLK = 1000
_EDGE_BLK = 3200


def _node_enc_body(x_ref, w1_ref, b1_ref, w2_ref, b2_ref, a_ref, b_ref,
                   h_ref, p_ref, q_ref):
    t = jnp.maximum(_dot(x_ref[...], w1_ref[...]) + b1_ref[...], 0.0)
    h = _dot(t, w2_ref[...]) + b2_ref[...]
    h_ref[...] = h
    p_ref[...] = _dot(h, a_ref[...])
    q_ref[...] = _dot(h, b_ref[...])


def _node_encode(x, w1, b1, w2, b2, a0, b0):
    grid = NN // _NODE_BLK
    kdim = x.shape[1]
    blk = lambda r, c: pl.BlockSpec((r, c), lambda i: (i, 0))
    wspec = lambda r, c: pl.BlockSpec((r, c), lambda i: (0, 0))
    out = jax.ShapeDtypeStruct((NN, H), jnp.float32)
    return pl.pallas_call(
        _node_enc_body,
        grid=(grid,),
        in_specs=[blk(_NODE_BLK, kdim), wspec(kdim, H), wspec(1, H),
                  wspec(H, H), wspec(1, H), wspec(H, H), wspec(H, H)],
        out_specs=[blk(_NODE_BLK, H)] * 3,
        out_shape=[out, out, out],
    )(x, w1, b1, w2, b2, a0, b0)


def _edge_enc_body(ea_ref, w1_ref, b1_ref, w2_ref, b2_ref, e_ref):
    t = jnp.maximum(_dot(ea_ref[...], w1_ref[...]) + b1_ref[...], 0.0)
    e_ref[...] = _dot(t, w2_ref[...]) + b2_ref[...]


def _edge_encode(ea, w1, b1, w2, b2):
    grid = NE // _EDGE_BLK
    kdim = ea.shape[1]
    blk = lambda r, c: pl.BlockSpec((r, c), lambda i: (i, 0))
    wspec = lambda r, c: pl.BlockSpec((r, c), lambda i: (0, 0))
    return pl.pallas_call(
        _edge_enc_body,
        grid=(grid,),
        in_specs=[blk(_EDGE_BLK, kdim), wspec(kdim, H), wspec(1, H),
                  wspec(H, H), wspec(1, H)],
        out_specs=blk(_EDGE_BLK, H),
        out_shape=jax.ShapeDtypeStruct((NE, H), jnp.float32),
    )(ea, w1, b1, w2, b2)


def _eproj_body(e_ref, c_ref, be_ref, o_ref):
    o_ref[...] = _dot(e_ref[...], c_ref[...]) + be_ref[...]


def _edge_project(e, c_l, be_l):
    grid = NE // _EDGE_BLK
    blk = lambda r, c: pl.BlockSpec((r, c), lambda i: (i, 0))
    wspec = lambda r, c: pl.BlockSpec((r, c), lambda i: (0, 0))
    return pl.pallas_call(
        _eproj_body,
        grid=(grid,),
        in_specs=[blk(_EDGE_BLK, H), wspec(H, H), wspec(1, H)],
        out_specs=blk(_EDGE_BLK, H),
        out_shape=jax.ShapeDtypeStruct((NE, H), jnp.float32),
    )(e, c_l, be_l)


def _node_upd_body(h_ref, g0_ref, g1_ref, wh_ref, wa_ref, bn_ref, a_ref,
                   b_ref, hn_ref, p_ref, q_ref):
    h = h_ref[...]
    agg = g0_ref[...] + g1_ref[...]
    u = jnp.maximum(_dot(h, wh_ref[...]) + _dot(agg, wa_ref[...]) + bn_ref[...], 0.0)
    hn = h + u
    hn_ref[...] = hn
    p_ref[...] = _dot(hn, a_ref[...])
    q_ref[...] = _dot(hn, b_ref[...])


def _node_update(h, g0, g1, wh, wa, bn, a_next, b_next):
    grid = NN // _NODE_BLK
    blk = lambda r, c: pl.BlockSpec((r, c), lambda i: (i, 0))
    wspec = lambda r, c: pl.BlockSpec((r, c), lambda i: (0, 0))
    out = jax.ShapeDtypeStruct((NN, H), jnp.float32)
    return pl.pallas_call(
        _node_upd_body,
        grid=(grid,),
        in_specs=[blk(_NODE_BLK, H)] * 3 + [wspec(H, H), wspec(H, H),
                                            wspec(1, H), wspec(H, H),
                                            wspec(H, H)],
        out_specs=[blk(_NODE_BLK, H)] * 3,
        out_shape=[out, out, out],
    )(h, g0, g1, wh, wa, bn, a_next, b_next)


def _minmax_body(c_ref, o_ref):
    c = c_ref[...]
    o_ref[0:1, :] = jnp.min(c, axis=0, keepdims=True)
    o_ref[1:2, :] = jnp.max(c, axis=0, keepdims=True)


def _coords_minmax(coords8):
    return pl.pallas_call(
        _minmax_body,
        out_shape=jax.ShapeDtypeStruct((2, 8), jnp.float32),
    )(coords8)


def _dec_body(c_ref, mm_ref, h_ref, bcd_ref, bcr_ref, w1c_ref, w1h_ref,
              b1_ref, w2_ref, b2_ref, w3_ref, b3_ref, w4_ref, b4_ref, o_ref):
    cmin = mm_ref[0:1, :]
    crange = jnp.maximum(mm_ref[1:2, :] - cmin, 1e-8)
    cn = (c_ref[...] - cmin) / crange
    t = jnp.maximum(_dot(cn, w1c_ref[...]) + _dot(h_ref[...], w1h_ref[...])
                    + b1_ref[...], 0.0)
    t = jnp.maximum(_dot(t, w2_ref[...]) + b2_ref[...], 0.0)
    t = jnp.maximum(_dot(t, w3_ref[...]) + b3_ref[...], 0.0)
    pred = _dot(t, w4_ref[...]) + b4_ref[...]
    col = lax.broadcasted_iota(jnp.int32, pred.shape, 1)
    factor = jnp.where(col < 2, 1.0 - bcd_ref[...], 1.0 - bcr_ref[...])
    o_ref[...] = pred * factor


def _decode(coords8, mm, h, bcd, bcr, w1c, w1h, b1, w2, b2, w3, b3, w4, b4):
    grid = NN // _NODE_BLK
    blk = lambda r, c: pl.BlockSpec((r, c), lambda i: (i, 0))
    wspec = lambda r, c: pl.BlockSpec((r, c), lambda i: (0, 0))
    return pl.pallas_call(
        _dec_body,
        grid=(grid,),
        in_specs=[blk(_NODE_BLK, 8), wspec(2, 8), blk(_NODE_BLK, H),
                  blk(_NODE_BLK, 1), blk(_NODE_BLK, 1), wspec(8, H),
                  wspec(H, H), wspec(1, H), wspec(H, H), wspec(1, H),
                  wspec(H, H), wspec(1, H), wspec(H, 8), wspec(1, 8)],
        out_specs=blk(_NODE_BLK, 8),
        out_shape=jax.ShapeDtypeStruct((NN, 8), jnp.float32),
    )(coords8, mm, h, bcd, bcr, w1c, w1h, b1, w2, b2, w3, b3, w4, b4)


_NC = 2
_NS = 16
_NT = _NC * _NS
_K = 128
_NCHUNK = NE // _K
_CPT = -(-_NCHUNK // _NT)
_NNP = 10240
_RPT = _NNP // _NS
_ZR = 128


def _mp_body(p_hbm, q_hbm, e_hbm, src_hbm, dst_hbm, out_hbm,
             idx_s, idx_d, pbuf, qbuf, ebuf, zbuf, agg_sh, sem_p, sem_q):
    cid = lax.axis_index("c")
    sid = lax.axis_index("s")
    wid = sid * _NC + cid

    zvec = jnp.zeros((16,), jnp.float32)

    def zrow(r, carry):
        for g in range(H // 16):
            zbuf[r, pl.ds(g * 16, 16)] = zvec
        return carry

    lax.fori_loop(0, _ZR, zrow, 0)
    for j in range(_RPT // _ZR):
        pltpu.sync_copy(zbuf, agg_sh.at[pl.ds(sid * _RPT + j * _ZR, _ZR)])
    plsc.subcore_barrier()

    def chunk(j, carry):
        c = wid + _NT * j

        @pl.when(c < _NCHUNK)
        def _():
            base = c * _K
            pltpu.sync_copy(src_hbm.at[pl.ds(base, _K)], idx_s)
            pltpu.sync_copy(dst_hbm.at[pl.ds(base, _K)], idx_d)
            cp_p = pltpu.make_async_copy(p_hbm.at[idx_s], pbuf, sem_p)
            cp_p.start()
            cp_q = pltpu.make_async_copy(q_hbm.at[idx_d], qbuf, sem_q)
            cp_q.start()
            pltpu.sync_copy(e_hbm.at[pl.ds(base, _K)], ebuf)
            cp_p.wait()
            cp_q.wait()

            def row(r, rc):
                for g in range(H // 16):
                    s = pl.ds(g * 16, 16)
                    ebuf[r, s] = jnp.maximum(
                        pbuf[r, s] + qbuf[r, s] + ebuf[r, s], 0.0)
                return rc

            lax.fori_loop(0, _K, row, 0)
            pltpu.sync_copy(ebuf, agg_sh.at[idx_d], add=True)

        return carry

    lax.fori_loop(0, _CPT, chunk, 0)
    plsc.subcore_barrier()
    pltpu.sync_copy(agg_sh.at[pl.ds(sid * _RPT, _RPT)],
                    out_hbm.at[cid, pl.ds(sid * _RPT, _RPT)])


@functools.cache
def _build_mp_call():
    return pl.kernel(
        _mp_body,
        out_type=jax.ShapeDtypeStruct((_NC, _NNP, H), jnp.float32),
        mesh=plsc.VectorSubcoreMesh(core_axis_name="c", subcore_axis_name="s"),
        compiler_params=pltpu.CompilerParams(use_tc_tiling_on_sc=False),
        scratch_types=[
            pltpu.VMEM((_K,), jnp.int32),
            pltpu.VMEM((_K,), jnp.int32),
            pltpu.VMEM((_K, H), jnp.float32),
            pltpu.VMEM((_K, H), jnp.float32),
            pltpu.VMEM((_K, H), jnp.float32),
            pltpu.VMEM((_ZR, H), jnp.float32),
            pltpu.VMEM_SHARED((_NNP, H), jnp.float32),
            pltpu.SemaphoreType.DMA,
            pltpu.SemaphoreType.DMA,
        ],
    )


def kernel(x, edge_attr, edge_index, coords, bc_disp, bc_rot, params):
    p = params
    src = edge_index[0]
    dst = edge_index[1]

    we = p['mp_We']
    a_all = we[:, 0:H, :]
    b_all = we[:, H:2 * H, :]
    c_all = we[:, 2 * H:3 * H, :]
    wn = p['mp_Wn']
    wh_all = wn[:, 0:H, :]
    wa_all = wn[:, H:2 * H, :]
    be_all = p['mp_be']
    bn_all = p['mp_bn']

    r1 = lambda v: v.reshape(1, -1)

    h, pproj, qproj = _node_encode(x, p['ne_W1'], r1(p['ne_b1']), p['ne_W2'],
                                   r1(p['ne_b2']), a_all[0], b_all[0])
    e = _edge_encode(edge_attr, p['ee_W1'], r1(p['ee_b1']), p['ee_W2'],
                     r1(p['ee_b2']))
    e_layers = [_edge_project(e, c_all[l], r1(be_all[l])) for l in range(NL)]

    mp_call = _build_mp_call()
    for l in range(NL):
        agg2 = mp_call(pproj, qproj, e_layers[l], src, dst)
        nxt = (l + 1) % NL
        h, pproj, qproj = _node_update(h, agg2[0, :NN], agg2[1, :NN], wh_all[l],
                                       wa_all[l], r1(bn_all[l]), a_all[nxt],
                                       b_all[nxt])

    coords8 = jnp.pad(coords, ((0, 0), (0, 8 - coords.shape[1])))
    mm = _coords_minmax(coords8)
    w1c = jnp.pad(p['dec_W1'][0:3, :], ((0, 5), (0, 0)))
    w1h = p['dec_W1'][3:, :]
    w4 = jnp.pad(p['dec_W4'], ((0, 0), (0, 8 - p['dec_W4'].shape[1])))
    b4 = jnp.pad(r1(p['dec_b4']), ((0, 0), (0, 8 - p['dec_b4'].shape[0])))
    pred8 = _decode(coords8, mm, h, bc_disp, bc_rot, w1c, w1h,
                    r1(p['dec_b1']), p['dec_W2'], r1(p['dec_b2']),
                    p['dec_W3'], r1(p['dec_b3']), w4, b4)
    return pred8[:, 0:3]


# speedup vs baseline: 3.9058x; 1.1062x over previous
import functools

import jax
import jax.numpy as jnp
from jax import lax
from jax.experimental import pallas as pl
from jax.experimental.pallas import tpu as pltpu
from jax.experimental.pallas import tpu_sc as plsc

H = 64
NL = 6
NN = 10000
NE = 320000


def _dot(a, b):
    return jnp.dot(a, b, precision=jax.lax.Precision.HIGHEST)

_NODE_BLK = 1000
_EDGE_BLK = 3200


def _node_enc_body(x_ref, w1_ref, b1_ref, w2_ref, b2_ref, a_ref, b_ref,
                   h_ref, p_ref, q_ref):
    t = jnp.maximum(_dot(x_ref[...], w1_ref[...]) + b1_ref[...], 0.0)
    h = _dot(t, w2_ref[...]) + b2_ref[...]
    h_ref[...] = h
    p_ref[...] = _dot(h, a_ref[...])
    q_ref[...] = _dot(h, b_ref[...])


def _node_encode(x, w1, b1, w2, b2, a0, b0):
    grid = NN // _NODE_BLK
    kdim = x.shape[1]
    blk = lambda r, c: pl.BlockSpec((r, c), lambda i: (i, 0))
    wspec = lambda r, c: pl.BlockSpec((r, c), lambda i: (0, 0))
    out = jax.ShapeDtypeStruct((NN, H), jnp.float32)
    return pl.pallas_call(
        _node_enc_body,
        grid=(grid,),
        in_specs=[blk(_NODE_BLK, kdim), wspec(kdim, H), wspec(1, H),
                  wspec(H, H), wspec(1, H), wspec(H, H), wspec(H, H)],
        out_specs=[blk(_NODE_BLK, H)] * 3,
        out_shape=[out, out, out],
    )(x, w1, b1, w2, b2, a0, b0)


def _edge_enc_body(ea_ref, w1_ref, b1_ref, t_ref):
    t_ref[...] = jnp.maximum(_dot(ea_ref[...], w1_ref[...]) + b1_ref[...], 0.0)


def _edge_encode(ea, w1, b1):
    grid = NE // _EDGE_BLK
    kdim = ea.shape[1]
    blk = lambda r, c: pl.BlockSpec((r, c), lambda i: (i, 0))
    wspec = lambda r, c: pl.BlockSpec((r, c), lambda i: (0, 0))
    return pl.pallas_call(
        _edge_enc_body,
        grid=(grid,),
        in_specs=[blk(_EDGE_BLK, kdim), wspec(kdim, H), wspec(1, H)],
        out_specs=blk(_EDGE_BLK, H),
        out_shape=jax.ShapeDtypeStruct((NE, H), jnp.float32),
    )(ea, w1, b1)


def _eproj_body(e_ref, c_ref, be_ref, o_ref):
    o_ref[...] = _dot(e_ref[...], c_ref[...]) + be_ref[...]


def _edge_project(e, c_l, be_l):
    grid = NE // _EDGE_BLK
    blk = lambda r, c: pl.BlockSpec((r, c), lambda i: (i, 0))
    wspec = lambda r, c: pl.BlockSpec((r, c), lambda i: (0, 0))
    return pl.pallas_call(
        _eproj_body,
        grid=(grid,),
        in_specs=[blk(_EDGE_BLK, H), wspec(H, H), wspec(1, H)],
        out_specs=blk(_EDGE_BLK, H),
        out_shape=jax.ShapeDtypeStruct((NE, H), jnp.float32),
    )(e, c_l, be_l)


def _wprep_body(w2_ref, ccat_ref, b2_ref, becat_ref, wc_ref, bc_ref):
    wc_ref[...] = _dot(w2_ref[...], ccat_ref[...])
    bc_ref[...] = _dot(b2_ref[...], ccat_ref[...]) + becat_ref[...]


def _weight_prep(w2, ccat, b2, becat):
    return pl.pallas_call(
        _wprep_body,
        out_shape=[jax.ShapeDtypeStruct((H, NL * H), jnp.float32),
                   jax.ShapeDtypeStruct((1, NL * H), jnp.float32)],
    )(w2, ccat, b2, becat)


def _node_upd_body(h_ref, g0_ref, g1_ref, wh_ref, wa_ref, bn_ref, a_ref,
                   b_ref, hn_ref, p_ref, q_ref):
    h = h_ref[...]
    agg = g0_ref[...] + g1_ref[...]
    u = jnp.maximum(_dot(h, wh_ref[...]) + _dot(agg, wa_ref[...]) + bn_ref[...], 0.0)
    hn = h + u
    hn_ref[...] = hn
    p_ref[...] = _dot(hn, a_ref[...])
    q_ref[...] = _dot(hn, b_ref[...])


def _node_update(h, g0, g1, wh, wa, bn, a_next, b_next):
    grid = NN // _NODE_BLK
    blk = lambda r, c: pl.BlockSpec((r, c), lambda i: (i, 0))
    wspec = lambda r, c: pl.BlockSpec((r, c), lambda i: (0, 0))
    out = jax.ShapeDtypeStruct((NN, H), jnp.float32)
    return pl.pallas_call(
        _node_upd_body,
        grid=(grid,),
        in_specs=[blk(_NODE_BLK, H)] * 3 + [wspec(H, H), wspec(H, H),
                                            wspec(1, H), wspec(H, H),
                                            wspec(H, H)],
        out_specs=[blk(_NODE_BLK, H)] * 3,
        out_shape=[out, out, out],
    )(h, g0, g1, wh, wa, bn, a_next, b_next)


def _minmax_body(c_ref, o_ref):
    c = c_ref[...]
    o_ref[0:1, :] = jnp.min(c, axis=0, keepdims=True)
    o_ref[1:2, :] = jnp.max(c, axis=0, keepdims=True)


def _coords_minmax(coords8):
    return pl.pallas_call(
        _minmax_body,
        out_shape=jax.ShapeDtypeStruct((2, 8), jnp.float32),
    )(coords8)


def _dec_body(c_ref, mm_ref, h_ref, bcd_ref, bcr_ref, w1c_ref, w1h_ref,
              b1_ref, w2_ref, b2_ref, w3_ref, b3_ref, w4_ref, b4_ref, o_ref):
    cmin = mm_ref[0:1, :]
    crange = jnp.maximum(mm_ref[1:2, :] - cmin, 1e-8)
    cn = (c_ref[...] - cmin) / crange
    t = jnp.maximum(_dot(cn, w1c_ref[...]) + _dot(h_ref[...], w1h_ref[...])
                    + b1_ref[...], 0.0)
    t = jnp.maximum(_dot(t, w2_ref[...]) + b2_ref[...], 0.0)
    t = jnp.maximum(_dot(t, w3_ref[...]) + b3_ref[...], 0.0)
    pred = _dot(t, w4_ref[...]) + b4_ref[...]
    col = lax.broadcasted_iota(jnp.int32, pred.shape, 1)
    factor = jnp.where(col < 2, 1.0 - bcd_ref[...], 1.0 - bcr_ref[...])
    o_ref[...] = pred * factor


def _decode(coords8, mm, h, bcd, bcr, w1c, w1h, b1, w2, b2, w3, b3, w4, b4):
    grid = NN // _NODE_BLK
    blk = lambda r, c: pl.BlockSpec((r, c), lambda i: (i, 0))
    wspec = lambda r, c: pl.BlockSpec((r, c), lambda i: (0, 0))
    return pl.pallas_call(
        _dec_body,
        grid=(grid,),
        in_specs=[blk(_NODE_BLK, 8), wspec(2, 8), blk(_NODE_BLK, H),
                  blk(_NODE_BLK, 1), blk(_NODE_BLK, 1), wspec(8, H),
                  wspec(H, H), wspec(1, H), wspec(H, H), wspec(1, H),
                  wspec(H, H), wspec(1, H), wspec(H, 8), wspec(1, 8)],
        out_specs=blk(_NODE_BLK, 8),
        out_shape=jax.ShapeDtypeStruct((NN, 8), jnp.float32),
    )(coords8, mm, h, bcd, bcr, w1c, w1h, b1, w2, b2, w3, b3, w4, b4)


_NC = 2
_NS = 16
_NT = _NC * _NS
_K = 128
_NCHUNK = NE // _K
_CPT = -(-_NCHUNK // _NT)
_NNP = 10240
_RPT = _NNP // _NS
_ZR = 128


def _mp_body(p_hbm, q_hbm, e_hbm, ei_hbm, out_hbm,
             idx_s, idx_d, pbuf, qbuf, ebuf, zbuf, agg_sh, sem_p, sem_q):
    cid = lax.axis_index("c")
    sid = lax.axis_index("s")
    wid = sid * _NC + cid

    zvec = jnp.zeros((16,), jnp.float32)

    def zrow(r, carry):
        for g in range(H // 16):
            zbuf[r, pl.ds(g * 16, 16)] = zvec
        return carry

    lax.fori_loop(0, _ZR, zrow, 0)
    for j in range(_RPT // _ZR):
        pltpu.sync_copy(zbuf, agg_sh.at[pl.ds(sid * _RPT + j * _ZR, _ZR)])
    plsc.subcore_barrier()

    def chunk(j, carry):
        c = wid + _NT * j

        @pl.when(c < _NCHUNK)
        def _():
            base = c * _K
            pltpu.sync_copy(ei_hbm.at[0, pl.ds(base, _K)], idx_s)
            pltpu.sync_copy(ei_hbm.at[1, pl.ds(base, _K)], idx_d)
            cp_p = pltpu.make_async_copy(p_hbm.at[idx_s], pbuf, sem_p)
            cp_p.start()
            cp_q = pltpu.make_async_copy(q_hbm.at[idx_d], qbuf, sem_q)
            cp_q.start()
            pltpu.sync_copy(e_hbm.at[pl.ds(base, _K)], ebuf)
            cp_p.wait()
            cp_q.wait()

            def row(r, rc):
                for g in range(H // 16):
                    s = pl.ds(g * 16, 16)
                    ebuf[r, s] = jnp.maximum(
                        pbuf[r, s] + qbuf[r, s] + ebuf[r, s], 0.0)
                return rc

            lax.fori_loop(0, _K, row, 0)
            pltpu.sync_copy(ebuf, agg_sh.at[idx_d], add=True)

        return carry

    lax.fori_loop(0, _CPT, chunk, 0)
    plsc.subcore_barrier()
    pltpu.sync_copy(agg_sh.at[pl.ds(sid * _RPT, _RPT)],
                    out_hbm.at[cid, pl.ds(sid * _RPT, _RPT)])


@functools.cache
def _build_mp_call():
    return pl.kernel(
        _mp_body,
        out_type=jax.ShapeDtypeStruct((_NC, _NNP, H), jnp.float32),
        mesh=plsc.VectorSubcoreMesh(core_axis_name="c", subcore_axis_name="s"),
        compiler_params=pltpu.CompilerParams(use_tc_tiling_on_sc=False),
        scratch_types=[
            pltpu.VMEM((_K,), jnp.int32),
            pltpu.VMEM((_K,), jnp.int32),
            pltpu.VMEM((_K, H), jnp.float32),
            pltpu.VMEM((_K, H), jnp.float32),
            pltpu.VMEM((_K, H), jnp.float32),
            pltpu.VMEM((_ZR, H), jnp.float32),
            pltpu.VMEM_SHARED((_NNP, H), jnp.float32),
            pltpu.SemaphoreType.DMA,
            pltpu.SemaphoreType.DMA,
        ],
    )


def kernel(x, edge_attr, edge_index, coords, bc_disp, bc_rot, params):
    p = params

    we = p['mp_We']
    a_all = we[:, 0:H, :]
    b_all = we[:, H:2 * H, :]
    c_all = we[:, 2 * H:3 * H, :]
    wn = p['mp_Wn']
    wh_all = wn[:, 0:H, :]
    wa_all = wn[:, H:2 * H, :]
    be_all = p['mp_be']
    bn_all = p['mp_bn']

    r1 = lambda v: v.reshape(1, -1)

    h, pproj, qproj = _node_encode(x, p['ne_W1'], r1(p['ne_b1']), p['ne_W2'],
                                   r1(p['ne_b2']), a_all[0], b_all[0])
    t = _edge_encode(edge_attr, p['ee_W1'], r1(p['ee_b1']))
    ccat = jnp.concatenate([c_all[l] for l in range(NL)], axis=1)
    becat = be_all.reshape(1, NL * H)
    wc, bc = _weight_prep(p['ee_W2'], ccat, r1(p['ee_b2']), becat)
    e_layers = [_edge_project(t, wc[:, l * H:(l + 1) * H],
                              bc[:, l * H:(l + 1) * H]) for l in range(NL)]

    mp_call = _build_mp_call()
    for l in range(NL):
        agg2 = mp_call(pproj, qproj, e_layers[l], edge_index)
        nxt = (l + 1) % NL
        h, pproj, qproj = _node_update(h, agg2[0, :NN], agg2[1, :NN], wh_all[l],
                                       wa_all[l], r1(bn_all[l]), a_all[nxt],
                                       b_all[nxt])

    coords8 = jnp.pad(coords, ((0, 0), (0, 8 - coords.shape[1])))
    mm = _coords_minmax(coords8)
    w1c = jnp.pad(p['dec_W1'][0:3, :], ((0, 5), (0, 0)))
    w1h = p['dec_W1'][3:, :]
    w4 = jnp.pad(p['dec_W4'], ((0, 0), (0, 8 - p['dec_W4'].shape[1])))
    b4 = jnp.pad(r1(p['dec_b4']), ((0, 0), (0, 8 - p['dec_b4'].shape[0])))
    pred8 = _decode(coords8, mm, h, bc_disp, bc_rot, w1c, w1h,
                    r1(p['dec_b1']), p['dec_W2'], r1(p['dec_b2']),
                    p['dec_W3'], r1(p['dec_b3']), w4, b4)
    return pred8[:, 0:3]
